# Initial kernel scaffold; baseline (speedup 1.0000x reference)
import jax
import jax.numpy as jnp
from jax.experimental import pallas as pl


def kernel(nodes, edges, globals_feat, params, receivers, senders, node_graph_idx, edge_graph_idx):
    raise NotImplementedError("write your pallas kernel here")


# speedup vs baseline: 1.0664x; 1.0664x over previous
import jax
import jax.numpy as jnp
from jax import lax
from jax.experimental import pallas as pl
from jax.experimental.pallas import tpu as pltpu
from jax.experimental.pallas import tpu_sc as plsc

_N, _E, _G = 2048, 32768, 8
_NW = 32
_F32 = jnp.float32


def _dot(a, b):
    return jnp.dot(a, b, preferred_element_type=_F32)


def _ln(h, s, b):
    mu = jnp.mean(h, axis=-1, keepdims=True)
    d = h - mu
    var = jnp.mean(d * d, axis=-1, keepdims=True)
    return d * lax.rsqrt(var + 1e-5) * s + b


def _onehot(idx, rows):
    return (idx[:, None] == lax.broadcasted_iota(jnp.int32, (rows, _G), 1)).astype(_F32)


def _glob_pre_body(g_ref, w0, b0, w1, b1, s, t, d_ref, vc_ref, ua_ref,
                   gd_o, gv_o, gu_o):
    x = g_ref[...]
    h = _dot(x, w0[...]) + b0[...]
    h = _dot(jax.nn.relu(h), w1[...]) + b1[...]
    h = _ln(h, s[...], t[...])
    gc = jnp.concatenate([x, h], axis=1)
    gd_o[...] = _dot(gc, d_ref[...])
    gv_o[...] = _dot(gc, vc_ref[...])
    gu_o[...] = _dot(gc, ua_ref[...])


def _glob_pre(gf, w0, b0, w1, b1, s, t, D, V1c, U1a):
    return pl.pallas_call(
        _glob_pre_body,
        out_shape=[jax.ShapeDtypeStruct((_G, 1024), _F32),
                   jax.ShapeDtypeStruct((_G, 1024), _F32),
                   jax.ShapeDtypeStruct((_G, 512), _F32)],
    )(gf, w0, b0, w1, b1, s, t, D, V1c, U1a)


def _node_pre_body(x_ref, w0, b0, w1, b1, s, t, a_ref, bm_ref, va_ref,
                   nr_o, ns_o, nv_o):
    x = x_ref[...]
    h = _dot(x, w0[...]) + b0[...]
    h = _dot(jax.nn.relu(h), w1[...]) + b1[...]
    h = _ln(h, s[...], t[...])
    nb = jnp.concatenate([x, h], axis=1)
    nr_o[...] = _dot(nb, a_ref[...])
    ns_o[...] = _dot(nb, bm_ref[...])
    nv_o[...] = _dot(nb, va_ref[...])


def _node_pre(nodes, w0, b0, w1, b1, s, t, A, Bm, V1a):
    blk = 256
    fx2 = lambda shp: pl.BlockSpec(shp, lambda i: (0, 0))
    fx1 = lambda shp: pl.BlockSpec(shp, lambda i: (0,))
    return pl.pallas_call(
        _node_pre_body,
        grid=(_N // blk,),
        in_specs=[pl.BlockSpec((blk, 512), lambda i: (i, 0)),
                  fx2((512, 512)), fx1((512,)), fx2((512, 512)), fx1((512,)),
                  fx1((512,)), fx1((512,)),
                  fx2((1024, 1024)), fx2((1024, 1024)), fx2((1024, 1024))],
        out_specs=[pl.BlockSpec((blk, 1024), lambda i: (i, 0))] * 3,
        out_shape=[jax.ShapeDtypeStruct((_N, 1024), _F32)] * 3,
    )(nodes, w0, b0, w1, b1, s, t, A, Bm, V1a)


def _edge_pre_body(x_ref, w0, b0, w1, b1, s, t, c_ref, gd_ref, b1e_ref, idx_ref,
                   ee_o):
    x = x_ref[...]
    h = _dot(x, w0[...]) + b0[...]
    h = _dot(jax.nn.relu(h), w1[...]) + b1[...]
    h = _ln(h, s[...], t[...])
    eb = jnp.concatenate([x, h], axis=1)
    oh = _onehot(idx_ref[0, 0, :], eb.shape[0])
    ee_o[...] = _dot(eb, c_ref[...]) + _dot(oh, gd_ref[...]) + b1e_ref[...]


def _edge_pre(edges, w0, b0, w1, b1, s, t, C, gd, b1e, egidx3):
    blk = 512
    fx2 = lambda shp: pl.BlockSpec(shp, lambda i: (0, 0))
    fx1 = lambda shp: pl.BlockSpec(shp, lambda i: (0,))
    return pl.pallas_call(
        _edge_pre_body,
        grid=(_E // blk,),
        in_specs=[pl.BlockSpec((blk, 128), lambda i: (i, 0)),
                  fx2((128, 128)), fx1((128,)), fx2((128, 128)), fx1((128,)),
                  fx1((128,)), fx1((128,)),
                  fx2((256, 1024)), fx2((_G, 1024)), fx1((1024,)),
                  pl.BlockSpec((1, 1, blk), lambda i: (i, 0, 0))],
        out_specs=pl.BlockSpec((blk, 1024), lambda i: (i, 0)),
        out_shape=jax.ShapeDtypeStruct((_E, 1024), _F32),
    )(edges, w0, b0, w1, b1, s, t, C, gd, b1e, egidx3)


def _edge_tail_body(ee_ref, g1_ref, g2_ref, w2, b2, s, t, idx_ref,
                    enew_o, enewt_o, eagg_o):
    h = ee_ref[...] + g1_ref[...] + g2_ref[...]
    y = _dot(jax.nn.relu(h), w2[...]) + b2[...]
    y = _ln(y, s[...], t[...])
    enew_o[...] = y
    enewt_o[...] = y.T
    oh = _onehot(idx_ref[0, 0, :], y.shape[0])
    part = lax.dot_general(oh, y, (((0,), (0,)), ((), ())),
                           preferred_element_type=_F32)

    @pl.when(pl.program_id(0) == 0)
    def _():
        eagg_o[...] = jnp.zeros(eagg_o.shape, eagg_o.dtype)

    eagg_o[...] += part


def _edge_tail(ee, g1, g2, W2e, b2e, s, t, egidx3):
    blk = 512
    fx2 = lambda shp: pl.BlockSpec(shp, lambda i: (0, 0))
    fx1 = lambda shp: pl.BlockSpec(shp, lambda i: (0,))
    return pl.pallas_call(
        _edge_tail_body,
        grid=(_E // blk,),
        in_specs=[pl.BlockSpec((blk, 1024), lambda i: (i, 0)),
                  pl.BlockSpec((blk, 1024), lambda i: (i, 0)),
                  pl.BlockSpec((blk, 1024), lambda i: (i, 0)),
                  fx2((1024, 512)), fx1((512,)), fx1((512,)), fx1((512,)),
                  pl.BlockSpec((1, 1, blk), lambda i: (i, 0, 0))],
        out_specs=[pl.BlockSpec((blk, 512), lambda i: (i, 0)),
                   pl.BlockSpec((512, blk), lambda i: (0, i)),
                   pl.BlockSpec((_G, 512), lambda i: (0, 0))],
        out_shape=[jax.ShapeDtypeStruct((_E, 512), _F32),
                   jax.ShapeDtypeStruct((512, _E), _F32),
                   jax.ShapeDtypeStruct((_G, 512), _F32)],
    )(ee, g1, g2, W2e, b2e, s, t, egidx3)


def _node_tail_body(nv_ref, a0_ref, vb, gv_ref, idx_ref, c1, v2, c2,
                    s, t, nnew_o, nagg_o):
    agg = a0_ref[...].T
    oh = _onehot(idx_ref[0, 0, :], agg.shape[0])
    h = nv_ref[...] + _dot(agg, vb[...]) + _dot(oh, gv_ref[...]) + c1[...]
    y = _dot(jax.nn.relu(h), v2[...]) + c2[...]
    y = _ln(y, s[...], t[...])
    nnew_o[...] = y
    part = lax.dot_general(oh, y, (((0,), (0,)), ((), ())),
                           preferred_element_type=_F32)

    @pl.when(pl.program_id(0) == 0)
    def _():
        nagg_o[...] = jnp.zeros(nagg_o.shape, nagg_o.dtype)

    nagg_o[...] += part


def _node_tail(nv, aggp, V1b, gv, ngidx3, c1, V2, c2, s, t):
    blk = 256
    fx2 = lambda shp: pl.BlockSpec(shp, lambda i: (0, 0))
    fx1 = lambda shp: pl.BlockSpec(shp, lambda i: (0,))
    nblk = _N // blk
    return pl.pallas_call(
        _node_tail_body,
        grid=(nblk,),
        in_specs=[pl.BlockSpec((blk, 1024), lambda i: (i, 0)),
                  pl.BlockSpec((512, blk), lambda i: (0, i)),
                  fx2((512, 1024)), fx2((_G, 1024)),
                  pl.BlockSpec((1, 1, blk), lambda i: (i, 0, 0)),
                  fx1((1024,)), fx2((1024, 512)), fx1((512,)),
                  fx1((512,)), fx1((512,))],
        out_specs=[pl.BlockSpec((blk, 512), lambda i: (i, 0)),
                   pl.BlockSpec((_G, 512), lambda i: (0, 0))],
        out_shape=[jax.ShapeDtypeStruct((_N, 512), _F32),
                   jax.ShapeDtypeStruct((_G, 512), _F32)],
    )(nv, aggp, V1b, gv, ngidx3, c1, V2, c2, s, t)


def _global_tail_body(gu_ref, na_ref, ea_ref, ub, uc, d1, u2, d2, s, t, gnew_o):
    h = gu_ref[...] + _dot(na_ref[...], ub[...]) + _dot(ea_ref[...], uc[...]) + d1[...]
    y = _dot(jax.nn.relu(h), u2[...]) + d2[...]
    gnew_o[...] = _ln(y, s[...], t[...])


def _global_tail(gu, nagg, eagg, U1b, U1c, d1, U2, d2, s, t):
    return pl.pallas_call(
        _global_tail_body,
        out_shape=jax.ShapeDtypeStruct((_G, 256), _F32),
    )(gu, nagg, eagg, U1b, U1c, d1, U2, d2, s, t)


_GCHUNK = 64
_SCHUNK = 128


def _sc_gather_body(nr_hbm, ns_hbm, recv_hbm, send_hbm, g1_hbm, g2_hbm,
                    idx_v, rows_v, sem):
    wid = lax.axis_index("s") * 2 + lax.axis_index("c")
    per = _E // _NW
    base = wid * per

    def body(k, carry):
        off = base + k * _GCHUNK
        pltpu.sync_copy(recv_hbm.at[pl.ds(off, _GCHUNK)], idx_v)
        pltpu.async_copy(nr_hbm.at[idx_v], rows_v, sem).wait()
        pltpu.sync_copy(rows_v, g1_hbm.at[pl.ds(off, _GCHUNK)])
        pltpu.sync_copy(send_hbm.at[pl.ds(off, _GCHUNK)], idx_v)
        pltpu.async_copy(ns_hbm.at[idx_v], rows_v, sem).wait()
        pltpu.sync_copy(rows_v, g2_hbm.at[pl.ds(off, _GCHUNK)])
        return carry

    lax.fori_loop(0, per // _GCHUNK, body, 0)


def _sc_gather(nr, ns, recv, send):
    mesh = plsc.VectorSubcoreMesh(core_axis_name="c", subcore_axis_name="s")
    f = pl.kernel(
        _sc_gather_body,
        out_type=[jax.ShapeDtypeStruct((_E, 1024), _F32)] * 2,
        mesh=mesh,
        scratch_types=[pltpu.VMEM((_GCHUNK,), jnp.int32),
                       pltpu.VMEM((_GCHUNK, 1024), _F32),
                       pltpu.SemaphoreType.DMA],
    )
    return f(nr, ns, recv, send)


_SBLK = 1024


def _sc_scatter_body(enewt_hbm, recv2_hbm, zeros_hbm, out_hbm, idx_v, rows_v,
                     acc_v):
    wid = lax.axis_index("s") * 2 + lax.axis_index("c")
    col = wid * 16
    pltpu.sync_copy(zeros_hbm, acc_v)
    col_iota = lax.broadcasted_iota(jnp.int32, (16,), 0)

    def body(k, carry):
        pltpu.sync_copy(recv2_hbm.at[pl.ds(k * 8, 8), :], idx_v)
        pltpu.sync_copy(enewt_hbm.at[pl.ds(col, 16), pl.ds(k * _SBLK, _SBLK)],
                        rows_v)

        def inner(sub, carry2):
            subv = jnp.full((16,), sub, jnp.int32)
            for e in range(_SCHUNK):
                ev = jnp.full((16,), e, jnp.int32)
                row = plsc.load_gather(idx_v, [subv, ev])
                x = plsc.load_gather(rows_v, [col_iota, subv * _SCHUNK + ev])
                plsc.addupdate_scatter(acc_v, [col_iota, row], x)
            return carry2

        lax.fori_loop(0, _SBLK // _SCHUNK, inner, 0)
        return carry

    lax.fori_loop(0, _E // _SBLK, body, 0)
    pltpu.sync_copy(acc_v, out_hbm.at[pl.ds(col, 16), :])


def _sc_scatter(enewt, recv2, zeros):
    mesh = plsc.VectorSubcoreMesh(core_axis_name="c", subcore_axis_name="s")
    f = pl.kernel(
        _sc_scatter_body,
        out_type=jax.ShapeDtypeStruct((512, _N), _F32),
        mesh=mesh,
        scratch_types=[pltpu.VMEM((8, _SCHUNK), jnp.int32),
                       pltpu.VMEM((16, _SBLK), _F32),
                       pltpu.VMEM((16, _N), _F32)],
        compiler_params=pltpu.CompilerParams(needs_layout_passes=False),
    )
    return f(enewt, recv2, zeros)


def kernel(nodes, edges, globals_feat, params, receivers, senders,
           node_graph_idx, edge_graph_idx):
    pn = params["node_encoder"]
    (wn0, bn0), (wn1, bn1) = pn["lin"]
    sn, tn = pn["ln"]
    pe = params["edge_encoder"]
    (we0, be0), (we1, be1) = pe["lin"]
    se, te = pe["ln"]
    pg = params["global_encoder"]
    (wg0, bg0), (wg1, bg1) = pg["lin"]
    sg, tg = pg["ln"]
    pm = params["edge_model"]
    (W1e, b1e), (W2e, b2e) = pm["lin"]
    sm, tm = pm["ln"]
    pv = params["node_model"]
    (V1, c1), (V2, c2) = pv["lin"]
    sv, tv = pv["ln"]
    pu = params["global_model"]
    (U1, d1), (U2, d2) = pu["lin"]
    su, tu = pu["ln"]

    A, Bm, C, D = W1e[:1024], W1e[1024:2048], W1e[2048:2304], W1e[2304:2560]
    V1a, V1b, V1c = V1[:1024], V1[1024:1536], V1[1536:1792]
    U1a, U1b, U1c = U1[:256], U1[256:768], U1[768:1280]

    recv = receivers.astype(jnp.int32)
    send = senders.astype(jnp.int32)
    egidx3 = edge_graph_idx.astype(jnp.int32).reshape(_E // 512, 1, 512)
    ngidx3 = node_graph_idx.astype(jnp.int32).reshape(_N // 256, 1, 256)

    gd, gv, gu = _glob_pre(globals_feat, wg0, bg0, wg1, bg1, sg, tg, D, V1c, U1a)
    nr, ns, nv = _node_pre(nodes, wn0, bn0, wn1, bn1, sn, tn, A, Bm, V1a)
    ee = _edge_pre(edges, we0, be0, we1, be1, se, te, C, gd, b1e, egidx3)
    g1, g2 = _sc_gather(nr, ns, recv, send)
    e_new, e_new_t, eagg = _edge_tail(ee, g1, g2, W2e, b2e, sm, tm, egidx3)
    zeros = jnp.zeros((16, _N), _F32)
    recv2 = recv.reshape(_E // _SCHUNK, _SCHUNK)
    aggp = _sc_scatter(e_new_t, recv2, zeros)
    n_new, nagg = _node_tail(nv, aggp, V1b, gv, ngidx3, c1, V2, c2, sv, tv)
    g_new = _global_tail(gu, nagg, eagg, U1b, U1c, d1, U2, d2, su, tu)
    return (n_new, e_new, g_new)


# speedup vs baseline: 2.9490x; 2.7655x over previous
import jax
import jax.numpy as jnp
from jax import lax
from jax.experimental import pallas as pl
from jax.experimental.pallas import tpu as pltpu
from jax.experimental.pallas import tpu_sc as plsc

_N, _E, _G = 2048, 32768, 8
_NW = 32
_F32 = jnp.float32


def _dot(a, b):
    return jnp.dot(a, b, preferred_element_type=_F32)


def _ln(h, s, b):
    mu = jnp.mean(h, axis=-1, keepdims=True)
    d = h - mu
    var = jnp.mean(d * d, axis=-1, keepdims=True)
    return d * lax.rsqrt(var + 1e-5) * s + b


def _onehot(idx, rows):
    return (idx[:, None] == lax.broadcasted_iota(jnp.int32, (rows, _G), 1)).astype(_F32)


def _glob_pre_body(g_ref, w0, b0, w1, b1, s, t, d_ref, vc_ref, ua_ref,
                   gd_o, gv_o, gu_o):
    x = g_ref[...]
    h = _dot(x, w0[...]) + b0[...]
    h = _dot(jax.nn.relu(h), w1[...]) + b1[...]
    h = _ln(h, s[...], t[...])
    gc = jnp.concatenate([x, h], axis=1)
    gd_o[...] = _dot(gc, d_ref[...])
    gv_o[...] = _dot(gc, vc_ref[...])
    gu_o[...] = _dot(gc, ua_ref[...])


def _glob_pre(gf, w0, b0, w1, b1, s, t, D, V1c, U1a):
    return pl.pallas_call(
        _glob_pre_body,
        out_shape=[jax.ShapeDtypeStruct((_G, 1024), _F32),
                   jax.ShapeDtypeStruct((_G, 1024), _F32),
                   jax.ShapeDtypeStruct((_G, 512), _F32)],
    )(gf, w0, b0, w1, b1, s, t, D, V1c, U1a)


def _node_pre_body(x_ref, w0, b0, w1, b1, s, t, a_ref, bm_ref, va_ref,
                   nr_o, ns_o, nv_o):
    x = x_ref[...]
    h = _dot(x, w0[...]) + b0[...]
    h = _dot(jax.nn.relu(h), w1[...]) + b1[...]
    h = _ln(h, s[...], t[...])
    nb = jnp.concatenate([x, h], axis=1)
    nr_o[...] = _dot(nb, a_ref[...])
    ns_o[...] = _dot(nb, bm_ref[...])
    nv_o[...] = _dot(nb, va_ref[...])


def _node_pre(nodes, w0, b0, w1, b1, s, t, A, Bm, V1a):
    blk = 256
    fx2 = lambda shp: pl.BlockSpec(shp, lambda i: (0, 0))
    fx1 = lambda shp: pl.BlockSpec(shp, lambda i: (0,))
    return pl.pallas_call(
        _node_pre_body,
        grid=(_N // blk,),
        in_specs=[pl.BlockSpec((blk, 512), lambda i: (i, 0)),
                  fx2((512, 512)), fx1((512,)), fx2((512, 512)), fx1((512,)),
                  fx1((512,)), fx1((512,)),
                  fx2((1024, 1024)), fx2((1024, 1024)), fx2((1024, 1024))],
        out_specs=[pl.BlockSpec((blk, 1024), lambda i: (i, 0))] * 3,
        out_shape=[jax.ShapeDtypeStruct((_N, 1024), _F32)] * 3,
    )(nodes, w0, b0, w1, b1, s, t, A, Bm, V1a)


def _edge_pre_body(x_ref, w0, b0, w1, b1, s, t, c_ref, gd_ref, b1e_ref, idx_ref,
                   ee_o):
    x = x_ref[...]
    h = _dot(x, w0[...]) + b0[...]
    h = _dot(jax.nn.relu(h), w1[...]) + b1[...]
    h = _ln(h, s[...], t[...])
    eb = jnp.concatenate([x, h], axis=1)
    oh = _onehot(idx_ref[0, 0, :], eb.shape[0])
    ee_o[...] = _dot(eb, c_ref[...]) + _dot(oh, gd_ref[...]) + b1e_ref[...]


def _edge_pre(edges, w0, b0, w1, b1, s, t, C, gd, b1e, egidx3):
    blk = 512
    fx2 = lambda shp: pl.BlockSpec(shp, lambda i: (0, 0))
    fx1 = lambda shp: pl.BlockSpec(shp, lambda i: (0,))
    return pl.pallas_call(
        _edge_pre_body,
        grid=(_E // blk,),
        in_specs=[pl.BlockSpec((blk, 128), lambda i: (i, 0)),
                  fx2((128, 128)), fx1((128,)), fx2((128, 128)), fx1((128,)),
                  fx1((128,)), fx1((128,)),
                  fx2((256, 1024)), fx2((_G, 1024)), fx1((1024,)),
                  pl.BlockSpec((1, 1, blk), lambda i: (i, 0, 0))],
        out_specs=pl.BlockSpec((blk, 1024), lambda i: (i, 0)),
        out_shape=jax.ShapeDtypeStruct((_E, 1024), _F32),
    )(edges, w0, b0, w1, b1, s, t, C, gd, b1e, egidx3)


def _edge_tail_body(ee_ref, g1_ref, g2_ref, w2, b2, s, t, idx_ref, recv_ref,
                    enew_o, eagg_o, agg_o):
    h = ee_ref[...] + g1_ref[...] + g2_ref[...]
    y = _dot(jax.nn.relu(h), w2[...]) + b2[...]
    y = _ln(y, s[...], t[...])
    enew_o[...] = y
    oh = _onehot(idx_ref[0, 0, :], y.shape[0])
    part = lax.dot_general(oh, y, (((0,), (0,)), ((), ())),
                           preferred_element_type=_F32)

    rids = recv_ref[0, 0, :]
    ohr = (rids[:, None] == lax.broadcasted_iota(jnp.int32, (rids.shape[0], _N), 1)
           ).astype(jnp.bfloat16)
    aggpart = lax.dot_general(ohr, y.astype(jnp.bfloat16),
                              (((0,), (0,)), ((), ())),
                              preferred_element_type=_F32)

    @pl.when(pl.program_id(0) == 0)
    def _():
        eagg_o[...] = jnp.zeros(eagg_o.shape, eagg_o.dtype)
        agg_o[...] = jnp.zeros(agg_o.shape, agg_o.dtype)

    eagg_o[...] += part
    agg_o[...] += aggpart


def _edge_tail(ee, g1, g2, W2e, b2e, s, t, egidx3, recv3):
    blk = 512
    fx2 = lambda shp: pl.BlockSpec(shp, lambda i: (0, 0))
    fx1 = lambda shp: pl.BlockSpec(shp, lambda i: (0,))
    return pl.pallas_call(
        _edge_tail_body,
        grid=(_E // blk,),
        in_specs=[pl.BlockSpec((blk, 1024), lambda i: (i, 0)),
                  pl.BlockSpec((blk, 1024), lambda i: (i, 0)),
                  pl.BlockSpec((blk, 1024), lambda i: (i, 0)),
                  fx2((1024, 512)), fx1((512,)), fx1((512,)), fx1((512,)),
                  pl.BlockSpec((1, 1, blk), lambda i: (i, 0, 0)),
                  pl.BlockSpec((1, 1, blk), lambda i: (i, 0, 0))],
        out_specs=[pl.BlockSpec((blk, 512), lambda i: (i, 0)),
                   pl.BlockSpec((_G, 512), lambda i: (0, 0)),
                   pl.BlockSpec((_N, 512), lambda i: (0, 0))],
        out_shape=[jax.ShapeDtypeStruct((_E, 512), _F32),
                   jax.ShapeDtypeStruct((_G, 512), _F32),
                   jax.ShapeDtypeStruct((_N, 512), _F32)],
    )(ee, g1, g2, W2e, b2e, s, t, egidx3, recv3)


def _node_tail_body(nv_ref, a0_ref, vb, gv_ref, idx_ref, c1, v2, c2,
                    s, t, nnew_o, nagg_o):
    agg = a0_ref[...]
    oh = _onehot(idx_ref[0, 0, :], agg.shape[0])
    h = nv_ref[...] + _dot(agg, vb[...]) + _dot(oh, gv_ref[...]) + c1[...]
    y = _dot(jax.nn.relu(h), v2[...]) + c2[...]
    y = _ln(y, s[...], t[...])
    nnew_o[...] = y
    part = lax.dot_general(oh, y, (((0,), (0,)), ((), ())),
                           preferred_element_type=_F32)

    @pl.when(pl.program_id(0) == 0)
    def _():
        nagg_o[...] = jnp.zeros(nagg_o.shape, nagg_o.dtype)

    nagg_o[...] += part


def _node_tail(nv, aggp, V1b, gv, ngidx3, c1, V2, c2, s, t):
    blk = 256
    fx2 = lambda shp: pl.BlockSpec(shp, lambda i: (0, 0))
    fx1 = lambda shp: pl.BlockSpec(shp, lambda i: (0,))
    nblk = _N // blk
    return pl.pallas_call(
        _node_tail_body,
        grid=(nblk,),
        in_specs=[pl.BlockSpec((blk, 1024), lambda i: (i, 0)),
                  pl.BlockSpec((blk, 512), lambda i: (i, 0)),
                  fx2((512, 1024)), fx2((_G, 1024)),
                  pl.BlockSpec((1, 1, blk), lambda i: (i, 0, 0)),
                  fx1((1024,)), fx2((1024, 512)), fx1((512,)),
                  fx1((512,)), fx1((512,))],
        out_specs=[pl.BlockSpec((blk, 512), lambda i: (i, 0)),
                   pl.BlockSpec((_G, 512), lambda i: (0, 0))],
        out_shape=[jax.ShapeDtypeStruct((_N, 512), _F32),
                   jax.ShapeDtypeStruct((_G, 512), _F32)],
    )(nv, aggp, V1b, gv, ngidx3, c1, V2, c2, s, t)


def _global_tail_body(gu_ref, na_ref, ea_ref, ub, uc, d1, u2, d2, s, t, gnew_o):
    h = gu_ref[...] + _dot(na_ref[...], ub[...]) + _dot(ea_ref[...], uc[...]) + d1[...]
    y = _dot(jax.nn.relu(h), u2[...]) + d2[...]
    gnew_o[...] = _ln(y, s[...], t[...])


def _global_tail(gu, nagg, eagg, U1b, U1c, d1, U2, d2, s, t):
    return pl.pallas_call(
        _global_tail_body,
        out_shape=jax.ShapeDtypeStruct((_G, 256), _F32),
    )(gu, nagg, eagg, U1b, U1c, d1, U2, d2, s, t)


_GCH = 32
_GPER = _E // _NW


def _sc_gather_body(nr_hbm, ns_hbm, recv_hbm, send_hbm, g1_hbm, g2_hbm,
                    idxr_v, idxs_v, buf0, buf1, gsem0, gsem1, wsem0, wsem1):
    wid = lax.axis_index("s") * 2 + lax.axis_index("c")
    base = wid * _GPER
    pltpu.sync_copy(recv_hbm.at[pl.ds(base, _GPER)], idxr_v)
    pltpu.sync_copy(send_hbm.at[pl.ds(base, _GPER)], idxs_v)

    nchunks = _GPER // _GCH
    bufs = (buf0, buf1)
    gsems = (gsem0, gsem1)
    wsems = (wsem0, wsem1)

    def plan(k):
        if k < nchunks:
            return nr_hbm, idxr_v, g1_hbm, k * _GCH
        return ns_hbm, idxs_v, g2_hbm, (k - nchunks) * _GCH

    def start_gather(k):
        src, idx, _, off = plan(k)
        return pltpu.async_copy(src.at[idx.at[pl.ds(off, _GCH)]],
                                bufs[k % 2], gsems[k % 2])

    total = 2 * nchunks
    g = start_gather(0)
    wprev = None
    for k in range(total):
        g.wait()
        if wprev is not None:
            wprev.wait()
        if k + 1 < total:
            g = start_gather(k + 1)
        _, _, dst, off = plan(k)
        wprev = pltpu.async_copy(bufs[k % 2], dst.at[pl.ds(base + off, _GCH)],
                                 wsems[k % 2])
    wprev.wait()


def _sc_gather(nr, ns, recv, send):
    mesh = plsc.VectorSubcoreMesh(core_axis_name="c", subcore_axis_name="s")
    f = pl.kernel(
        _sc_gather_body,
        out_type=[jax.ShapeDtypeStruct((_E, 1024), _F32)] * 2,
        mesh=mesh,
        scratch_types=[pltpu.VMEM((_GPER,), jnp.int32),
                       pltpu.VMEM((_GPER,), jnp.int32),
                       pltpu.VMEM((_GCH, 1024), _F32),
                       pltpu.VMEM((_GCH, 1024), _F32),
                       pltpu.SemaphoreType.DMA, pltpu.SemaphoreType.DMA,
                       pltpu.SemaphoreType.DMA, pltpu.SemaphoreType.DMA],
    )
    return f(nr, ns, recv, send)


def kernel(nodes, edges, globals_feat, params, receivers, senders,
           node_graph_idx, edge_graph_idx):
    pn = params["node_encoder"]
    (wn0, bn0), (wn1, bn1) = pn["lin"]
    sn, tn = pn["ln"]
    pe = params["edge_encoder"]
    (we0, be0), (we1, be1) = pe["lin"]
    se, te = pe["ln"]
    pg = params["global_encoder"]
    (wg0, bg0), (wg1, bg1) = pg["lin"]
    sg, tg = pg["ln"]
    pm = params["edge_model"]
    (W1e, b1e), (W2e, b2e) = pm["lin"]
    sm, tm = pm["ln"]
    pv = params["node_model"]
    (V1, c1), (V2, c2) = pv["lin"]
    sv, tv = pv["ln"]
    pu = params["global_model"]
    (U1, d1), (U2, d2) = pu["lin"]
    su, tu = pu["ln"]

    A, Bm, C, D = W1e[:1024], W1e[1024:2048], W1e[2048:2304], W1e[2304:2560]
    V1a, V1b, V1c = V1[:1024], V1[1024:1536], V1[1536:1792]
    U1a, U1b, U1c = U1[:256], U1[256:768], U1[768:1280]

    recv = receivers.astype(jnp.int32)
    send = senders.astype(jnp.int32)
    recv3 = recv.reshape(_E // 512, 1, 512)
    egidx3 = edge_graph_idx.astype(jnp.int32).reshape(_E // 512, 1, 512)
    ngidx3 = node_graph_idx.astype(jnp.int32).reshape(_N // 256, 1, 256)

    gd, gv, gu = _glob_pre(globals_feat, wg0, bg0, wg1, bg1, sg, tg, D, V1c, U1a)
    nr, ns, nv = _node_pre(nodes, wn0, bn0, wn1, bn1, sn, tn, A, Bm, V1a)
    ee = _edge_pre(edges, we0, be0, we1, be1, se, te, C, gd, b1e, egidx3)
    g1, g2 = _sc_gather(nr, ns, recv, send)
    e_new, eagg, agg = _edge_tail(ee, g1, g2, W2e, b2e, sm, tm, egidx3, recv3)
    n_new, nagg = _node_tail(nv, agg, V1b, gv, ngidx3, c1, V2, c2, sv, tv)
    g_new = _global_tail(gu, nagg, eagg, U1b, U1c, d1, U2, d2, su, tu)
    return (n_new, e_new, g_new)
